# baseline (speedup 1.0000x reference)
import functools

import jax
import jax.numpy as jnp
from jax import lax
from jax.experimental import pallas as pl
from jax.experimental.pallas import tpu as pltpu
from jax.experimental.pallas import tpu_sc as plsc

NUM_EPOCHS = 1000000
BATCH = 16384
NC = 2
NS = 16
L = 16
NW = NC * NS
BPW = BATCH // NW
GROUPS = BPW // L
CHUNK = 128


def _rsqrt(n2):
    i = plsc.bitcast(n2, jnp.int32)
    i = jnp.int32(0x5F3759DF) - (i >> 1)
    y = plsc.bitcast(i, jnp.float32)
    for _ in range(3):
        y = y * (1.5 - 0.5 * n2 * y * y)
    return y


def _mult(a, b):
    a0, a1, a2, a3 = a
    b0, b1, b2, b3 = b
    return (
        a3 * b0 + a0 * b3 + a1 * b2 - a2 * b1,
        a3 * b1 - a0 * b2 + a1 * b3 + a2 * b0,
        a3 * b2 + a0 * b1 - a1 * b0 + a2 * b3,
        a3 * b3 - a0 * b0 - a1 * b1 - a2 * b2,
    )


def _transform(v, r):
    v0, v1, v2 = v
    r0, r1, r2, r3 = r
    n12 = r0 + r0
    n2 = r1 + r1
    n = r2 + r2
    n11 = r3 * n12
    n10 = r3 * n2
    n9 = r3 * n
    n8 = r0 * n12
    n7 = r0 * n2
    n6 = r0 * n
    n5 = r1 * n2
    n4 = r1 * n
    n3 = r2 * n
    t0 = v0 * (1.0 - n5 - n3) + v1 * (n7 - n9) + v2 * (n6 + n10)
    t1 = v0 * (n7 + n9) + v1 * (1.0 - n8 - n3) + v2 * (n4 - n11)
    t2 = v0 * (n6 - n10) + v1 * (n4 + n11) + v2 * (1.0 - n8 - n5)
    return t0, t1, t2


_MESH = plsc.VectorSubcoreMesh(core_axis_name="c", subcore_axis_name="s")


@functools.partial(
    pl.kernel,
    mesh=_MESH,
    compiler_params=pltpu.CompilerParams(
        needs_layout_passes=False, use_tc_tiling_on_sc=False),
    out_type=[
        jax.ShapeDtypeStruct((BATCH * 4,), jnp.float32),
        jax.ShapeDtypeStruct((BATCH * 3,), jnp.float32),
        jax.ShapeDtypeStruct((NW * L,), jnp.float32),
    ],
    scratch_types=[
        pltpu.VMEM((BPW,), jnp.int32),
        pltpu.VMEM((BPW * 4,), jnp.int32),
        pltpu.VMEM((BPW * 4,), jnp.int32),
        pltpu.VMEM((BPW * 3,), jnp.int32),
        pltpu.VMEM((BPW * 3,), jnp.int32),
        pltpu.VMEM((BPW * 4,), jnp.float32),
        pltpu.VMEM((BPW * 4,), jnp.float32),
        pltpu.VMEM((BPW * 3,), jnp.float32),
        pltpu.VMEM((BPW * 3,), jnp.float32),
        pltpu.VMEM((BPW * 3,), jnp.float32),
        pltpu.VMEM((BPW * 4,), jnp.float32),
        pltpu.VMEM((BPW * 3,), jnp.float32),
        pltpu.VMEM((L,), jnp.float32),
        pltpu.VMEM((L,), jnp.int32),
        pltpu.VMEM((L,), jnp.float32),
        pltpu.VMEM((BPW * 4,), jnp.float32),
        pltpu.VMEM((BPW * 3,), jnp.float32),
        pltpu.VMEM((L,), jnp.float32),
        pltpu.SemaphoreType.DMA,
    ],
)
def _rigid_sc(epoch, acs, gyr, mag, g, speed, quat,
              out_q, out_s, out_p,
              idx_v, iq1_v, iq2_v, is1_v, is2_v,
              q1_v, q2_v, s1_v, s2_v,
              acs_v, gyr_v, mag_v, g_v, bidx_v, bnd_v,
              oq_v, os_v, p_v, sem):
    wid = lax.axis_index("s") * NC + lax.axis_index("c")
    base = wid * BPW
    iota = lax.iota(jnp.int32, L)

    pltpu.sync_copy(epoch.at[pl.ds(base, BPW)], idx_v)

    def idx_body(grp, _):
        sl = pl.ds(grp * L, L)
        v = idx_v[sl]
        vm = v - 1
        v4 = v * 4
        vm4 = vm * 4
        v3 = v * 3
        vm3 = vm * 3
        for c in range(4):
            iq1_v[pl.ds(c * BPW + grp * L, L)] = v4 + c
            iq2_v[pl.ds(c * BPW + grp * L, L)] = vm4 + c
        for c in range(3):
            is1_v[pl.ds(c * BPW + grp * L, L)] = v3 + c
            is2_v[pl.ds(c * BPW + grp * L, L)] = vm3 + c
        return 0

    lax.fori_loop(0, GROUPS, idx_body, 0)
    bidx_v[...] = jnp.where(
        iota < 3, iota + 3,
        jnp.where(iota < 6, iota + (3 * (NUM_EPOCHS - 1) - 3), 3))

    copies = []
    for m in range(BPW * 4 // CHUNK):
        sl = pl.ds(m * CHUNK, CHUNK)
        copies.append(pltpu.async_copy(quat.at[iq1_v.at[sl]], q1_v.at[sl], sem))
        copies.append(pltpu.async_copy(quat.at[iq2_v.at[sl]], q2_v.at[sl], sem))
    for m in range(BPW * 3 // CHUNK):
        sl = pl.ds(m * CHUNK, CHUNK)
        copies.append(pltpu.async_copy(speed.at[is1_v.at[sl]], s1_v.at[sl], sem))
        copies.append(pltpu.async_copy(speed.at[is2_v.at[sl]], s2_v.at[sl], sem))
    copies.append(pltpu.async_copy(speed.at[bidx_v], bnd_v, sem))
    copies.append(pltpu.async_copy(acs.at[pl.ds(base * 3, BPW * 3)], acs_v, sem))
    copies.append(pltpu.async_copy(gyr.at[pl.ds(base * 4, BPW * 4)], gyr_v, sem))
    copies.append(pltpu.async_copy(mag.at[pl.ds(base * 3, BPW * 3)], mag_v, sem))
    copies.append(pltpu.async_copy(g, g_v, sem))
    for c in copies:
        c.wait()

    g_s = g_v[...]
    zero = jnp.zeros((L,), jnp.float32)

    def body(grp, accs):
        macc, qacc, aacc = accs
        rows = grp * L + iota

        def soa(ref, c):
            return ref[pl.ds(c * BPW + grp * L, L)]

        def aos(ref, n, c):
            return plsc.load_gather(ref, [rows * n + c])

        q1 = tuple(soa(q1_v, c) for c in range(4))
        q2 = tuple(soa(q2_v, c) for c in range(4))
        sp1 = tuple(soa(s1_v, c) for c in range(3))
        sp2 = tuple(soa(s2_v, c) for c in range(3))
        ac = tuple(aos(acs_v, 3, c) for c in range(3))
        gy = tuple(aos(gyr_v, 4, c) for c in range(4))
        mg = tuple(aos(mag_v, 3, c) for c in range(3))

        r1 = _rsqrt(q1[0] * q1[0] + q1[1] * q1[1] + q1[2] * q1[2] + q1[3] * q1[3])
        qt1 = tuple(c * r1 for c in q1)
        r2 = _rsqrt(q2[0] * q2[0] + q2[1] * q2[1] + q2[2] * q2[2] + q2[3] * q2[3])
        qt2 = tuple(c * r2 for c in q2)
        rm = _rsqrt(mg[0] * mg[0] + mg[1] * mg[1] + mg[2] * mg[2])
        ort = tuple(c * rm for c in mg)

        t0, t1, t2 = _transform(ort, qt1)
        d0 = t0 - 1.0
        macc = macc + d0 * d0 + t1 * t1 + t2 * t2

        m = _mult(qt2, gy)
        for c in range(4):
            dq = m[c] - qt1[c]
            qacc = qacc + dq * dq

        a0, a1, a2 = _transform(ac, qt1)
        e0 = a0 - (sp1[0] - sp2[0])
        e1 = a1 - (sp1[1] - sp2[1])
        e2 = (a2 + g_s) - (sp1[2] - sp2[2])
        aacc = aacc + e0 * e0 + e1 * e1 + e2 * e2

        for c in range(4):
            plsc.store_scatter(oq_v, [rows * 4 + c], qt1[c])
        for c in range(3):
            plsc.store_scatter(os_v, [rows * 3 + c], sp1[c])
        return macc, qacc, aacc

    macc, qacc, aacc = lax.fori_loop(0, GROUPS, body, (zero, zero, zero))

    bv = bnd_v[...]
    bsum = jnp.sum(jnp.where(iota < 6, bv * bv, 0.0))

    m_s = jnp.sum(macc)
    q_s = jnp.sum(qacc)
    a_s = jnp.sum(aacc)
    p_v[...] = jnp.where(
        iota == 0, m_s,
        jnp.where(iota == 1, q_s,
                  jnp.where(iota == 2, a_s,
                            jnp.where(iota == 3, bsum, 0.0))))

    pltpu.sync_copy(oq_v, out_q.at[pl.ds(base * 4, BPW * 4)])
    pltpu.sync_copy(os_v, out_s.at[pl.ds(base * 3, BPW * 3)])
    pltpu.sync_copy(p_v, out_p.at[pl.ds(wid * L, L)])


def kernel(epoch_input, acs_input, gyr_input, mag_input, g, speed, quat):
    g16 = jnp.broadcast_to(jnp.reshape(g, (1,)), (L,))
    outq, outs, outp = _rigid_sc(
        epoch_input.astype(jnp.int32),
        acs_input.reshape(-1), gyr_input.reshape(-1), mag_input.reshape(-1),
        g16, speed.reshape(-1), quat.reshape(-1))
    qt1 = outq.reshape(BATCH, 4)
    speed1 = outs.reshape(BATCH, 3)
    partials = outp.reshape(NW, L)
    mag_loss = jnp.sum(partials[:, 0]) / (BATCH * 3) / 10.0
    quat_loss = jnp.sum(partials[:, 1]) / (BATCH * 4)
    acs_loss = jnp.sum(partials[:, 2]) / (BATCH * 3) * 1e-05
    bnd_loss = partials[0, 3] / 3.0 * 0.01
    total_loss = mag_loss + quat_loss + acs_loss + bnd_loss
    return qt1, speed1, total_loss


# speedup vs baseline: 36.0599x; 36.0599x over previous
import functools

import jax
import jax.numpy as jnp
from jax import lax
from jax.experimental import pallas as pl
from jax.experimental.pallas import tpu as pltpu
from jax.experimental.pallas import tpu_sc as plsc

NUM_EPOCHS = 1000000
BATCH = 16384
NT = NUM_EPOCHS + 1
NC = 2
NS = 16
L = 16
NW = NC * NS
BPW = BATCH // NW
GROUPS = BPW // L
CHUNK = 128


def _rsqrt(n2):
    i = plsc.bitcast(n2, jnp.int32)
    i = jnp.int32(0x5F3759DF) - (i >> 1)
    y = plsc.bitcast(i, jnp.float32)
    for _ in range(3):
        y = y * (1.5 - 0.5 * n2 * y * y)
    return y


def _mult(a, b):
    a0, a1, a2, a3 = a
    b0, b1, b2, b3 = b
    return (
        a3 * b0 + a0 * b3 + a1 * b2 - a2 * b1,
        a3 * b1 - a0 * b2 + a1 * b3 + a2 * b0,
        a3 * b2 + a0 * b1 - a1 * b0 + a2 * b3,
        a3 * b3 - a0 * b0 - a1 * b1 - a2 * b2,
    )


def _transform(v, r):
    v0, v1, v2 = v
    r0, r1, r2, r3 = r
    n12 = r0 + r0
    n2 = r1 + r1
    n = r2 + r2
    n11 = r3 * n12
    n10 = r3 * n2
    n9 = r3 * n
    n8 = r0 * n12
    n7 = r0 * n2
    n6 = r0 * n
    n5 = r1 * n2
    n4 = r1 * n
    n3 = r2 * n
    t0 = v0 * (1.0 - n5 - n3) + v1 * (n7 - n9) + v2 * (n6 + n10)
    t1 = v0 * (n7 + n9) + v1 * (1.0 - n8 - n3) + v2 * (n4 - n11)
    t2 = v0 * (n6 - n10) + v1 * (n4 + n11) + v2 * (1.0 - n8 - n5)
    return t0, t1, t2


_MESH = plsc.VectorSubcoreMesh(core_axis_name="c", subcore_axis_name="s")


@functools.partial(
    pl.kernel,
    mesh=_MESH,
    compiler_params=pltpu.CompilerParams(
        needs_layout_passes=False, use_tc_tiling_on_sc=False),
    out_type=[
        jax.ShapeDtypeStruct((BATCH * 4,), jnp.float32),
        jax.ShapeDtypeStruct((BATCH * 3,), jnp.float32),
        jax.ShapeDtypeStruct((NW * L,), jnp.float32),
    ],
    scratch_types=[
        pltpu.VMEM((BPW,), jnp.int32),
        pltpu.VMEM((BPW * 4,), jnp.int32),
        pltpu.VMEM((BPW * 4,), jnp.int32),
        pltpu.VMEM((BPW * 4,), jnp.float32),
        pltpu.VMEM((BPW * 4,), jnp.float32),
        pltpu.VMEM((BPW * 3,), jnp.float32),
        pltpu.VMEM((BPW * 3,), jnp.float32),
        pltpu.VMEM((BPW * 3,), jnp.float32),
        pltpu.VMEM((BPW * 4,), jnp.float32),
        pltpu.VMEM((BPW * 3,), jnp.float32),
        pltpu.VMEM((L,), jnp.float32),
        pltpu.VMEM((L,), jnp.int32),
        pltpu.VMEM((L,), jnp.float32),
        pltpu.VMEM((BPW * 4,), jnp.float32),
        pltpu.VMEM((L,), jnp.float32),
        pltpu.SemaphoreType.DMA,
    ],
)
def _rigid_sc(epoch, acs, gyr, mag, g, speed, quat,
              out_q, out_s, out_p,
              idx_v, ib1_v, ib2_v,
              q1_v, q2_v, s1_v, s2_v,
              acs_v, gyr_v, mag_v, g_v, bidx_v, bnd_v,
              oq_v, p_v, sem):
    wid = lax.axis_index("s") * NC + lax.axis_index("c")
    base = wid * BPW
    iota = lax.iota(jnp.int32, L)

    pltpu.sync_copy(epoch.at[pl.ds(base, BPW)], idx_v)

    def idx_body(grp, _):
        sl = pl.ds(grp * L, L)
        v = idx_v[sl]
        vm = v - 1
        for c in range(4):
            ib1_v[pl.ds(c * BPW + grp * L, L)] = v + (c * NT)
            ib2_v[pl.ds(c * BPW + grp * L, L)] = vm + (c * NT)
        return 0

    lax.fori_loop(0, GROUPS, idx_body, 0)
    bidx_v[...] = jnp.where(
        iota < 3, iota * NT + 1,
        jnp.where(iota < 6, (iota - 3) * NT + (NUM_EPOCHS - 1), 1))

    copies = []
    for m in range(BPW * 4 // CHUNK):
        sl = pl.ds(m * CHUNK, CHUNK)
        copies.append(pltpu.async_copy(quat.at[ib1_v.at[sl]], q1_v.at[sl], sem))
        copies.append(pltpu.async_copy(quat.at[ib2_v.at[sl]], q2_v.at[sl], sem))
    for m in range(BPW * 3 // CHUNK):
        sl = pl.ds(m * CHUNK, CHUNK)
        copies.append(pltpu.async_copy(speed.at[ib1_v.at[sl]], s1_v.at[sl], sem))
        copies.append(pltpu.async_copy(speed.at[ib2_v.at[sl]], s2_v.at[sl], sem))
    copies.append(pltpu.async_copy(speed.at[bidx_v], bnd_v, sem))
    for c in range(3):
        csl = pl.ds(c * BPW, BPW)
        copies.append(pltpu.async_copy(
            acs.at[pl.ds(c * BATCH + base, BPW)], acs_v.at[csl], sem))
        copies.append(pltpu.async_copy(
            mag.at[pl.ds(c * BATCH + base, BPW)], mag_v.at[csl], sem))
    for c in range(4):
        copies.append(pltpu.async_copy(
            gyr.at[pl.ds(c * BATCH + base, BPW)],
            gyr_v.at[pl.ds(c * BPW, BPW)], sem))
    copies.append(pltpu.async_copy(g, g_v, sem))
    for c in copies:
        c.wait()

    g_s = g_v[...]
    zero = jnp.zeros((L,), jnp.float32)

    def body(grp, accs):
        macc, qacc, aacc = accs

        def soa(ref, c):
            return ref[pl.ds(c * BPW + grp * L, L)]

        q1 = tuple(soa(q1_v, c) for c in range(4))
        q2 = tuple(soa(q2_v, c) for c in range(4))
        sp1 = tuple(soa(s1_v, c) for c in range(3))
        sp2 = tuple(soa(s2_v, c) for c in range(3))
        ac = tuple(soa(acs_v, c) for c in range(3))
        gy = tuple(soa(gyr_v, c) for c in range(4))
        mg = tuple(soa(mag_v, c) for c in range(3))

        r1 = _rsqrt(q1[0] * q1[0] + q1[1] * q1[1] + q1[2] * q1[2] + q1[3] * q1[3])
        qt1 = tuple(c * r1 for c in q1)
        r2 = _rsqrt(q2[0] * q2[0] + q2[1] * q2[1] + q2[2] * q2[2] + q2[3] * q2[3])
        qt2 = tuple(c * r2 for c in q2)
        rm = _rsqrt(mg[0] * mg[0] + mg[1] * mg[1] + mg[2] * mg[2])
        ort = tuple(c * rm for c in mg)

        t0, t1, t2 = _transform(ort, qt1)
        d0 = t0 - 1.0
        macc = macc + d0 * d0 + t1 * t1 + t2 * t2

        m = _mult(qt2, gy)
        for c in range(4):
            dq = m[c] - qt1[c]
            qacc = qacc + dq * dq

        a0, a1, a2 = _transform(ac, qt1)
        e0 = a0 - (sp1[0] - sp2[0])
        e1 = a1 - (sp1[1] - sp2[1])
        e2 = (a2 + g_s) - (sp1[2] - sp2[2])
        aacc = aacc + e0 * e0 + e1 * e1 + e2 * e2

        for c in range(4):
            oq_v[pl.ds(c * BPW + grp * L, L)] = qt1[c]
        return macc, qacc, aacc

    macc, qacc, aacc = lax.fori_loop(0, GROUPS, body, (zero, zero, zero))

    bv = bnd_v[...]
    bsum = jnp.sum(jnp.where(iota < 6, bv * bv, 0.0))

    m_s = jnp.sum(macc)
    q_s = jnp.sum(qacc)
    a_s = jnp.sum(aacc)
    p_v[...] = jnp.where(
        iota == 0, m_s,
        jnp.where(iota == 1, q_s,
                  jnp.where(iota == 2, a_s,
                            jnp.where(iota == 3, bsum, 0.0))))

    for c in range(4):
        pltpu.sync_copy(oq_v.at[pl.ds(c * BPW, BPW)],
                        out_q.at[pl.ds(c * BATCH + base, BPW)])
    for c in range(3):
        pltpu.sync_copy(s1_v.at[pl.ds(c * BPW, BPW)],
                        out_s.at[pl.ds(c * BATCH + base, BPW)])
    pltpu.sync_copy(p_v, out_p.at[pl.ds(wid * L, L)])


def kernel(epoch_input, acs_input, gyr_input, mag_input, g, speed, quat):
    g16 = jnp.broadcast_to(jnp.reshape(g, (1,)), (L,))
    outq, outs, outp = _rigid_sc(
        epoch_input.astype(jnp.int32),
        acs_input.T.reshape(-1), gyr_input.T.reshape(-1),
        mag_input.T.reshape(-1), g16,
        speed.T.reshape(-1), quat.T.reshape(-1))
    qt1 = outq.reshape(4, BATCH).T
    speed1 = outs.reshape(3, BATCH).T
    partials = outp.reshape(NW, L)
    mag_loss = jnp.sum(partials[:, 0]) / (BATCH * 3) / 10.0
    quat_loss = jnp.sum(partials[:, 1]) / (BATCH * 4)
    acs_loss = jnp.sum(partials[:, 2]) / (BATCH * 3) * 1e-05
    bnd_loss = partials[0, 3] / 3.0 * 0.01
    total_loss = mag_loss + quat_loss + acs_loss + bnd_loss
    return qt1, speed1, total_loss
